# baseline (speedup 1.0000x reference)
import functools

import jax
import jax.numpy as jnp
from jax import lax
from jax.experimental import pallas as pl
from jax.experimental.pallas import tpu as pltpu
from jax.experimental.pallas import tpu_sc as plsc

VOCAB = 1000
EMBED = 1000
BATCH = 1024
SEQ = 20
EMBED_PAD = 1024

UNIT = 16
RING = 4


@functools.lru_cache(maxsize=None)
def _build(batch, seq, embed, embed_pad):
    info = plsc.get_sparse_core_info()
    nc, ns = info.num_cores, info.num_subcores
    nw = nc * ns
    bpw = batch // nw
    assert bpw * nw == batch
    rpw = bpw * seq
    nunit = rpw // UNIT
    assert nunit * UNIT == rpw
    ring_rows = RING * UNIT

    mesh = plsc.VectorSubcoreMesh(core_axis_name="c", subcore_axis_name="s")

    offs = list(range(0, embed - 16, 16)) + [embed - 16]

    @functools.partial(
        pl.kernel,
        mesh=mesh,
        out_type=jax.ShapeDtypeStruct((batch, seq, embed), jnp.float32),
        scratch_types=[
            pltpu.VMEM((rpw,), jnp.int32),
            pltpu.VMEM((ring_rows, embed_pad), jnp.float32),
            pltpu.VMEM((2, seq, embed), jnp.float32),
            [pltpu.SemaphoreType.DMA] * RING,
            [pltpu.SemaphoreType.DMA] * 2,
        ],
    )
    def emb(x_hbm, table_hbm, out_hbm, idx_v, ring_v, buf_v, sg, sw):
        wid = lax.axis_index("s") * nc + lax.axis_index("c")
        b0 = wid * bpw
        pltpu.sync_copy(x_hbm.at[pl.ds(b0 * seq, rpw)], idx_v)

        def gather(u):
            return pltpu.async_copy(
                table_hbm.at[idx_v.at[pl.ds(u * UNIT, UNIT)]],
                ring_v.at[pl.ds((u % RING) * UNIT, UNIT)],
                sg[u % RING],
            )

        def realign(k, p):
            def row(s, carry):
                gr = jnp.bitwise_and(seq * k + s, ring_rows - 1)
                for o in offs:
                    buf_v[p, s, pl.ds(o, 16)] = ring_v[gr, pl.ds(o, 16)]
                return carry

            lax.fori_loop(0, seq, row, 0)

        pend_g = [gather(u) for u in range(RING)]
        issued = RING
        waited = 0
        pend_w = [None] * bpw
        for k in range(bpw):
            last_u = (seq * k + seq - 1) // UNIT
            while waited <= last_u:
                pend_g[waited].wait()
                waited += 1
            if k >= 2:
                pend_w[k - 2].wait()
            p = k % 2
            realign(k, p)
            pend_w[k] = pltpu.async_copy(buf_v.at[p], out_hbm.at[b0 + k], sw[p])
            freed = (seq * (k + 1)) // UNIT
            while issued < nunit and issued < freed + RING:
                pend_g.append(gather(issued))
                issued += 1
        pend_w[bpw - 2].wait()
        pend_w[bpw - 1].wait()

    return emb


def kernel(x, table):
    emb = _build(BATCH, SEQ, EMBED, EMBED_PAD)
    table_pad = jnp.pad(table, ((0, 0), (0, EMBED_PAD - EMBED)))
    out = emb(x.reshape(-1), table_pad)
    return out
